# Initial kernel scaffold; baseline (speedup 1.0000x reference)
import jax
import jax.numpy as jnp
from jax.experimental import pallas as pl


def kernel(x0, x1, x2, patches_idx, kernels):
    raise NotImplementedError("write your pallas kernel here")


# speedup vs baseline: 3.7007x; 3.7007x over previous
import functools
from math import factorial

import numpy as np
import jax
import jax.numpy as jnp
from jax import lax
from jax.experimental import pallas as pl
from jax.experimental.pallas import tpu as pltpu
from jax.experimental.pallas import tpu_sc as plsc

L_MAX = 2
L_MAX_OUT = 2


def _f(n):
    return float(factorial(int(round(n))))


def _su2_cg(j1, m1, j2, m2, j3, m3):
    if m3 != m1 + m2:
        return 0.0
    vmin = int(max(-j1 + j2 + m3, -j1 + m1, 0))
    vmax = int(min(j2 + j3 + m1, j3 - j1 + j2, j3 + m3))
    C = ((2.0 * j3 + 1.0) * _f(j3 + j1 - j2) * _f(j3 - j1 + j2) * _f(j1 + j2 - j3) / _f(j1 + j2 + j3 + 1)
         * _f(j3 + m3) * _f(j3 - m3) / (_f(j1 + m1) * _f(j1 - m1) * _f(j2 + m2) * _f(j2 - m2))) ** 0.5
    S = 0.0
    for v in range(vmin, vmax + 1):
        S += (-1.0) ** (v + j2 + m2) / _f(v) * _f(j2 + j3 + m1 - v) * _f(j1 - m1 + v) / _f(j3 - j1 + j2 - v) / _f(j3 + m3 - v) / _f(v + j1 - j2 - m3)
    return C * S


def _su2_cg_tensor(j1, j2, j3):
    mat = np.zeros((2 * j1 + 1, 2 * j2 + 1, 2 * j3 + 1))
    for m1 in range(-j1, j1 + 1):
        for m2 in range(-j2, j2 + 1):
            m3 = m1 + m2
            if abs(m3) <= j3:
                mat[j1 + m1, j2 + m2, j3 + m3] = _su2_cg(j1, m1, j2, m2, j3, m3)
    return mat


def _real_to_complex(l):
    q = np.zeros((2 * l + 1, 2 * l + 1), dtype=np.complex128)
    s2 = np.sqrt(2.0)
    for m in range(-l, 0):
        q[l + m, l + abs(m)] = 1.0 / s2
        q[l + m, l - abs(m)] = -1j / s2
    q[l, l] = 1.0
    for m in range(1, l + 1):
        q[l + m, l + abs(m)] = (-1.0) ** m / s2
        q[l + m, l - abs(m)] = 1j * (-1.0) ** m / s2
    return ((-1j) ** l) * q


def _so3_cg(l1, l2, l3):
    Q1 = _real_to_complex(l1)
    Q2 = _real_to_complex(l2)
    Q3 = _real_to_complex(l3)
    C = _su2_cg_tensor(l1, l2, l3).astype(np.complex128)
    C = np.einsum('ij,kl,mn,ikn->jlm', Q1, Q2, np.conj(Q3.T), C)
    return np.real(C).astype(np.float32)


def _build_postmap():
    split_size = [(2 * l + 1) * (L_MAX + 1 - l) for l in range(L_MAX + 1)]
    coffs = np.cumsum([0, 8, 24, 40])
    soffs = np.cumsum([0] + split_size)
    N = 14 * 72
    y = np.eye(N, dtype=np.float64).reshape(N, 1, 14, 72)
    bb, vv = N, 1
    out = {str(j): [] for j in range(L_MAX + 1)}
    y_cg = []
    for i, l in enumerate([0, 1, 2]):
        yl = y[..., int(coffs[i]):int(coffs[i + 1])]
        for j in range(L_MAX + 1):
            yij = yl[:, :, int(soffs[j]):int(soffs[j + 1]), :]
            yij = yij.reshape(bb, vv, 2 * j + 1, L_MAX + 1 - j, 2 * l + 1, -1)
            yij = np.transpose(yij, (0, 1, 2, 4, 3, 5))
            yij = yij.reshape(bb, vv, 2 * j + 1, 2 * l + 1, -1)
            if l == 0:
                out[str(j)].append(yij[:, :, :, 0, :])
            elif j == 0:
                out[str(l)].append(yij[:, :, 0, :, :])
            else:
                y_cg.append((j, l, yij))
    for (j, l, t) in y_cg:
        for J in range(abs(j - l), min(j + l, L_MAX_OUT) + 1):
            cg = _so3_cg(j, l, J).astype(np.float64)
            out[str(J)].append(np.einsum('mnJ,bvmnc->bvJc', cg, t))
    mats = []
    for J in range(L_MAX_OUT + 1):
        o = np.concatenate(out[str(J)], axis=-1)
        mats.append(o.reshape(N, -1))
    W = np.concatenate(mats, axis=1)
    return W.reshape(14, 72, 712).astype(np.float32)


_W_POSTMAP = _build_postmap()
_CH_OUT = (48, 264, 400)


_NC, _NS = 2, 16
_NW = _NC * _NS
_CHUNK = 128
_NBUF = 4


def _sc_gather_body(nchunk, table_hbm, idx_hbm, out_hbm, idx_v, rows_v, *sems):
    wid = lax.axis_index("s") * _NC + lax.axis_index("c")
    rows_per_w = nchunk * _CHUNK
    base = wid * rows_per_w
    pltpu.sync_copy(idx_hbm.at[wid], idx_v)

    def start(c, slot):
        pltpu.make_async_copy(
            table_hbm.at[idx_v.at[c]], rows_v.at[slot], sems[slot]).start()

    def wait(c, slot):
        pltpu.make_async_copy(
            table_hbm.at[idx_v.at[c]], rows_v.at[slot], sems[slot]).wait()

    for slot in range(_NBUF):
        start(slot, slot)

    def group(g, _):
        for slot in range(_NBUF):
            c = g * _NBUF + slot
            wait(c, slot)
            pltpu.sync_copy(rows_v.at[slot],
                            out_hbm.at[pl.ds(base + c * _CHUNK, _CHUNK)])
            nxt = c + _NBUF

            @pl.when(nxt < nchunk)
            def _():
                start(nxt, slot)
        return 0

    lax.fori_loop(0, nchunk // _NBUF, group, 0)


def _make_sc_gather(n_rows_out, nchunk):
    mesh = plsc.VectorSubcoreMesh(core_axis_name="c", subcore_axis_name="s")
    scratch = [
        pltpu.VMEM((nchunk, _CHUNK), jnp.int32),
        pltpu.VMEM((_NBUF, _CHUNK, 72), jnp.float32),
    ] + [pltpu.SemaphoreType.DMA] * _NBUF
    return functools.partial(
        pl.kernel,
        out_type=jax.ShapeDtypeStruct((n_rows_out, 72), jnp.float32),
        mesh=mesh,
        scratch_types=scratch,
        compiler_params=pltpu.CompilerParams(use_tc_tiling_on_sc=False),
    )(functools.partial(_sc_gather_body, nchunk))


_VB = 256
_SUB = 8


def _tc_body(g_ref, k_ref, w_ref, out_ref, y_scr):
    def chunk(s, _):
        gs = g_ref[pl.ds(s * _SUB, _SUB)]
        ks = k_ref[pl.ds(s * _SUB, _SUB)]
        for i in range(14):
            yi = jnp.sum(gs * ks[:, :, i:i + 1], axis=1)
            y_scr[i, pl.ds(s * _SUB, _SUB), :] = yi
        return 0

    lax.fori_loop(0, _VB // _SUB, chunk, 0)
    acc = jnp.zeros((_VB, 712), jnp.float32)
    for i in range(14):
        acc += jnp.dot(y_scr[i], w_ref[i],
                       preferred_element_type=jnp.float32)
    out_ref[...] = acc


def _make_tc_conv(n_points):
    grid = n_points // _VB
    return pl.pallas_call(
        _tc_body,
        grid=(grid,),
        in_specs=[
            pl.BlockSpec((_VB, 32, 72), lambda i: (i, 0, 0)),
            pl.BlockSpec((_VB, 32, 14), lambda i: (i, 0, 0)),
            pl.BlockSpec((14, 72, 712), lambda i: (0, 0, 0)),
        ],
        out_specs=pl.BlockSpec((_VB, 712), lambda i: (i, 0)),
        out_shape=jax.ShapeDtypeStruct((n_points, 712), jnp.float32),
        scratch_shapes=[pltpu.VMEM((14, _VB, 72), jnp.float32)],
    )


def kernel(x0, x1, x2, patches_idx, kernels):
    b, n = x0.shape[0], x0.shape[1]
    v, p = patches_idx.shape[1], patches_idx.shape[2]
    signal = jnp.concatenate(
        [t.reshape(b, n, -1) for t in (x0, x1, x2)], axis=-1)
    table = signal.reshape(b * n, 72)

    flat_idx = (patches_idx[..., 0].astype(jnp.int32) * n
                + patches_idx[..., 1].astype(jnp.int32))
    n_rows = b * v * p
    rows_per_w = n_rows // _NW
    nchunk = rows_per_w // _CHUNK
    idx3 = flat_idx.reshape(_NW, nchunk, _CHUNK)

    gathered = _make_sc_gather(n_rows, nchunk)(table, idx3)

    n_points = b * v
    out = _make_tc_conv(n_points)(
        gathered.reshape(n_points, p, 72),
        kernels.reshape(n_points, p, 14),
        jnp.asarray(_W_POSTMAP),
    )

    o0 = out[:, :_CH_OUT[0]].reshape(b, v, 1, 48)
    o1 = out[:, _CH_OUT[0]:_CH_OUT[0] + _CH_OUT[1]].reshape(b, v, 3, 88)
    o2 = out[:, _CH_OUT[0] + _CH_OUT[1]:].reshape(b, v, 5, 80)
    return (o0, o1, o2)
